# baseline (speedup 1.0000x reference)
import jax
import jax.numpy as jnp
from jax import lax
from jax.experimental import pallas as pl
from jax.experimental.pallas import tpu as pltpu
from jax.experimental.pallas import tpu_sc as plsc

VOCAB = 1000000
MAX_LEN = 200
EMB = 64
BATCH = 4096

NC = 2
NS = 16
NW = NC * NS
SEQ_PER_W = BATCH // NW
NCH = 5
CH = MAX_LEN // NCH
LANES = 16


def _body(x_hbm, tab_hbm, pos_hbm, out_hbm, idx_v, pos_v, buf, g0, g1, s0, s1):
    c = lax.axis_index("c")
    s = lax.axis_index("s")
    wid = s * NC + c

    pltpu.sync_copy(x_hbm.at[wid], idx_v)
    pltpu.sync_copy(pos_hbm, pos_v)

    gsems = (g0, g1)
    ssems = (s0, s1)

    def start_gather(q, b):
        for h in range(NCH):
            pltpu.async_copy(
                tab_hbm.at[idx_v.at[NCH * q + h]],
                buf.at[b, pl.ds(h * CH, CH)],
                gsems[b],
            )

    def wait_gather(b):
        for h in range(NCH):
            pltpu.make_async_copy(
                tab_hbm.at[idx_v.at[0]],
                buf.at[b, pl.ds(h * CH, CH)],
                gsems[b],
            ).wait()

    def start_store(q, b):
        pltpu.async_copy(buf.at[b], out_hbm.at[wid * SEQ_PER_W + q], ssems[b])

    def wait_store(b):
        pltpu.make_async_copy(buf.at[b], out_hbm.at[0], ssems[b]).wait()

    def add_pos(b):
        def row(i, _):
            for j in range(EMB // LANES):
                sl = pl.ds(j * LANES, LANES)
                buf[b, i, sl] += pos_v[i, sl]
            return 0

        lax.fori_loop(0, MAX_LEN, row, 0, unroll=2)

    start_gather(0, 0)
    start_gather(1, 1)

    def step(i, _):
        q0 = 2 * i
        wait_gather(0)
        add_pos(0)
        start_store(q0, 0)
        wait_gather(1)
        add_pos(1)
        start_store(q0 + 1, 1)

        @pl.when(i < SEQ_PER_W // 2 - 1)
        def _():
            wait_store(0)
            start_gather(q0 + 2, 0)
            wait_store(1)
            start_gather(q0 + 3, 1)

        return 0

    lax.fori_loop(0, SEQ_PER_W // 2, step, 0)
    wait_store(0)
    wait_store(1)


_mesh = plsc.VectorSubcoreMesh(core_axis_name="c", subcore_axis_name="s")

_emb = pl.kernel(
    _body,
    out_type=jax.ShapeDtypeStruct((BATCH, MAX_LEN, EMB), jnp.float32),
    mesh=_mesh,
    compiler_params=pltpu.CompilerParams(use_tc_tiling_on_sc=False),
    scratch_types=[
        pltpu.VMEM((SEQ_PER_W * NCH, CH), jnp.int32),
        pltpu.VMEM((MAX_LEN, EMB), jnp.float32),
        pltpu.VMEM((2, MAX_LEN, EMB), jnp.float32),
        pltpu.SemaphoreType.DMA,
        pltpu.SemaphoreType.DMA,
        pltpu.SemaphoreType.DMA,
        pltpu.SemaphoreType.DMA,
    ],
)


@jax.jit
def kernel(x, token_table, pos_table):
    xi = x.astype(jnp.int32).reshape(NW, SEQ_PER_W * NCH, CH)
    return _emb(xi, token_table, pos_table)
